# baseline (speedup 1.0000x reference)
import functools
import math

import jax
import jax.numpy as jnp
from jax.experimental import pallas as pl
from jax.experimental.pallas import tpu as pltpu

D_MODEL = 256
GRID_DIM = 30
HW = GRID_DIM * GRID_DIM


def _body(x_ref, io_ref, pair_ref, o_ref, pos_scr, ge_scr):
    b = pl.program_id(0)
    g = pl.program_id(1)

    @pl.when(jnp.logical_and(b == 0, g == 0))
    def _init():
        dim = D_MODEL // 2
        r = jax.lax.broadcasted_iota(jnp.int32, (HW, 2 * dim), 0)
        c = jax.lax.broadcasted_iota(jnp.int32, (HW, 2 * dim), 1)
        pos = jnp.where(c < dim, r // GRID_DIM, r % GRID_DIM).astype(jnp.float32)
        cl = c % dim
        freq = jnp.exp((cl - cl % 2).astype(jnp.float32) * (-math.log(10000.0) / dim))
        angle = pos * freq
        pos_scr[...] = jnp.where(cl % 2 == 0, jnp.sin(angle), jnp.cos(angle))
        for gg in range(9):
            ge_scr[gg, 0:64] = io_ref[gg % 2, :]
            ge_scr[gg, 64:128] = pair_ref[gg // 2, :]

    xb = x_ref[0, 0]
    o_ref[0, 0, :, 0:256] = xb[:, 0:256] + pos_scr[...]
    o_ref[0, 0, :, 256:384] = xb[:, 256:384] + ge_scr[g, :][None, :]


@jax.jit
def kernel(x, io_table, pair_table):
    B, G, H, W, C = x.shape
    xf = x.reshape(B, G, H * W, C)
    out = pl.pallas_call(
        _body,
        grid=(B, G),
        in_specs=[
            pl.BlockSpec((1, 1, H * W, C), lambda b, g: (b, g, 0, 0)),
            pl.BlockSpec(memory_space=pltpu.VMEM),
            pl.BlockSpec(memory_space=pltpu.VMEM),
        ],
        out_specs=pl.BlockSpec((1, 1, H * W, C), lambda b, g: (b, g, 0, 0)),
        out_shape=jax.ShapeDtypeStruct((B, G, H * W, C), x.dtype),
        scratch_shapes=[
            pltpu.VMEM((HW, 2 * (D_MODEL // 2)), jnp.float32),
            pltpu.VMEM((9, D_MODEL // 2), jnp.float32),
        ],
    )(xf, io_table, pair_table)
    return out.reshape(B, G, H, W, C)
